# baseline (speedup 1.0000x reference)
import functools
import math

import jax
import jax.numpy as jnp
from jax.experimental import pallas as pl
from jax.experimental.pallas import tpu as pltpu

BLOCK = 128
NEG_INF = -1e30


def _mask_body(q_ref, k_ref, mask_ref, *, nb, block, scale, min_retain,
               max_retain):
    d = q_ref.shape[-1]
    qh = q_ref[0]
    kh = k_ref[0]
    qp = qh.reshape(nb, block, d).mean(axis=1)
    kp = kh.reshape(nb, block, d).mean(axis=1)
    s = jax.lax.dot_general(qp, kp, (((1,), (1,)), ((), ())),
                            preferred_element_type=jnp.float32) * scale
    m = jnp.max(s, axis=-1, keepdims=True)
    e = jnp.exp(s - m)
    p = e / jnp.sum(e, axis=-1, keepdims=True)

    col_ids = jax.lax.broadcasted_iota(jnp.int32, (nb, nb), 1)
    rank = jnp.zeros((nb, nb), jnp.float32)
    for j in range(nb):
        col = p[:, j:j + 1]
        gt = jnp.sum((p > col).astype(jnp.float32), axis=-1, keepdims=True)
        if j > 0:
            eq = jnp.sum((p[:, :j] == col).astype(jnp.float32), axis=-1,
                         keepdims=True)
        else:
            eq = jnp.zeros_like(gt)
        rank = rank + (gt + eq) * (col_ids == j).astype(jnp.float32)

    cum = jnp.zeros((nb, nb), jnp.float32)
    for i in range(nb):
        le = (rank <= float(i)).astype(jnp.float32)
        ci = jnp.sum(p * le, axis=-1, keepdims=True)
        cum = cum + ci * (col_ids == i).astype(jnp.float32)

    thr = 0.95 * cum[:, nb - 1:nb]
    kcnt = jnp.sum((cum < thr).astype(jnp.float32), axis=-1, keepdims=True)
    kk = jnp.clip(kcnt, float(min_retain), float(max_retain))
    mask_ref[0] = (rank < kk).astype(jnp.int32)


def _attn_body(mask_smem, q_ref, k_ref, v_ref, out_ref, *, nb, block):
    h = pl.program_id(0)
    i = pl.program_id(1)
    base = (h * nb + i) * nb

    qb = q_ref[0]

    es, ms, ls = [], [], []
    for j in range(nb):
        kj = k_ref[0, j * block:(j + 1) * block, :]
        s = jax.lax.dot_general(qb, kj, (((1,), (1,)), ((), ())),
                                preferred_element_type=jnp.float32)
        mj = jnp.max(s, axis=-1, keepdims=True)
        e = jnp.exp(s - mj)
        es.append(e)
        ms.append(mj)
        ls.append(jnp.sum(e, axis=-1, keepdims=True))

    keeps = [mask_smem[base + j] > 0 for j in range(nb)]
    mm = [jnp.where(keeps[j], ms[j], NEG_INF) for j in range(nb)]
    while len(mm) > 1:
        mm = [jnp.maximum(mm[a], mm[a + 1]) if a + 1 < len(mm) else mm[a]
              for a in range(0, len(mm), 2)]
    M = mm[0]
    betas = [jnp.where(keeps[j], jnp.exp(ms[j] - M), 0.0)
             for j in range(nb)]
    l = betas[0] * ls[0]
    for j in range(1, nb):
        l = l + betas[j] * ls[j]

    outs = []
    for j in range(nb):
        vj = v_ref[0, j * block:(j + 1) * block, :]
        pj = (es[j] * betas[j]).astype(jnp.bfloat16)
        outs.append(jax.lax.dot_general(pj, vj,
                                        (((1,), (0,)), ((), ())),
                                        preferred_element_type=jnp.float32))
    while len(outs) > 1:
        outs = [outs[a] + outs[a + 1] if a + 1 < len(outs) else outs[a]
                for a in range(0, len(outs), 2)]

    out_ref[0] = outs[0] / l


@jax.jit
def kernel(q, k, v):
    B, H, S, d = q.shape
    nb = S // BLOCK
    BH = B * H
    scale = 1.0 / math.sqrt(d)
    min_retain = max(1, int(nb * 0.05))
    max_retain = max(1, int(nb * 0.7))

    qf = q.reshape(BH, S, d)
    kf = k.reshape(BH, S, d)
    vf = v.reshape(BH, S, d)

    mask = pl.pallas_call(
        functools.partial(_mask_body, nb=nb, block=BLOCK, scale=scale,
                          min_retain=min_retain, max_retain=max_retain),
        grid=(BH,),
        in_specs=[
            pl.BlockSpec((1, S, d), lambda h: (h, 0, 0)),
            pl.BlockSpec((1, S, d), lambda h: (h, 0, 0)),
        ],
        out_specs=pl.BlockSpec((1, nb, nb), lambda h: (h, 0, 0)),
        out_shape=jax.ShapeDtypeStruct((BH, nb, nb), jnp.int32),
        compiler_params=pltpu.CompilerParams(
            dimension_semantics=("arbitrary",)),
    )(qf, kf)

    mask_flat = mask.reshape(-1)
    qs = (qf * scale).astype(jnp.bfloat16)
    kb = kf.astype(jnp.bfloat16)
    vb = vf.astype(jnp.bfloat16)

    grid_spec = pltpu.PrefetchScalarGridSpec(
        num_scalar_prefetch=1,
        grid=(BH, nb),
        in_specs=[
            pl.BlockSpec((1, BLOCK, d), lambda h, i, mask: (h, i, 0)),
            pl.BlockSpec((1, S, d), lambda h, i, mask: (h, 0, 0)),
            pl.BlockSpec((1, S, d), lambda h, i, mask: (h, 0, 0)),
        ],
        out_specs=pl.BlockSpec((1, BLOCK, d), lambda h, i, mask: (h, i, 0)),
    )
    out = pl.pallas_call(
        functools.partial(_attn_body, nb=nb, block=BLOCK),
        grid_spec=grid_spec,
        out_shape=jax.ShapeDtypeStruct((BH, S, d), jnp.float32),
        compiler_params=pltpu.CompilerParams(
            dimension_semantics=("parallel", "arbitrary")),
    )(mask_flat, qs, kb, vb)

    return out.reshape(B, H, S, d)


# speedup vs baseline: 1.1781x; 1.1781x over previous
import functools
import math

import jax
import jax.numpy as jnp
from jax.experimental import pallas as pl
from jax.experimental.pallas import tpu as pltpu

BLOCK = 128


def _mask_body(q_ref, k_ref, meta_ref, *, nb, block, scale, min_retain,
               max_retain):
    d = q_ref.shape[-1]
    qh = q_ref[0]
    kh = k_ref[0]
    qp = qh.reshape(nb, block, d).mean(axis=1)
    kp = kh.reshape(nb, block, d).mean(axis=1)
    s = jax.lax.dot_general(qp, kp, (((1,), (1,)), ((), ())),
                            preferred_element_type=jnp.float32) * scale
    m = jnp.max(s, axis=-1, keepdims=True)
    e = jnp.exp(s - m)
    p = e / jnp.sum(e, axis=-1, keepdims=True)

    col_ids = jax.lax.broadcasted_iota(jnp.int32, (nb, nb), 1)
    colf = col_ids.astype(jnp.float32)
    rank = jnp.zeros((nb, nb), jnp.float32)
    for j in range(nb):
        col = p[:, j:j + 1]
        gt = jnp.sum((p > col).astype(jnp.float32), axis=-1, keepdims=True)
        if j > 0:
            eq = jnp.sum((p[:, :j] == col).astype(jnp.float32), axis=-1,
                         keepdims=True)
        else:
            eq = jnp.zeros_like(gt)
        rank = rank + (gt + eq) * (col_ids == j).astype(jnp.float32)

    cum = jnp.zeros((nb, nb), jnp.float32)
    for i in range(nb):
        le = (rank <= float(i)).astype(jnp.float32)
        ci = jnp.sum(p * le, axis=-1, keepdims=True)
        cum = cum + ci * (col_ids == i).astype(jnp.float32)

    thr = 0.95 * cum[:, nb - 1:nb]
    kcnt = jnp.sum((cum < thr).astype(jnp.float32), axis=-1, keepdims=True)
    kk = jnp.clip(kcnt, float(min_retain), float(max_retain))
    kept = (rank < kk).astype(jnp.float32)

    riota = jax.lax.broadcasted_iota(jnp.int32, (nb, nb), 0)
    upper = (riota < col_ids).astype(jnp.float32)
    pos = jax.lax.dot_general(kept, upper, (((1,), (0,)), ((), ())),
                              preferred_element_type=jnp.float32)
    lastid = jnp.max(jnp.where(kept > 0.0, colf, -1.0), axis=-1,
                     keepdims=True)
    meta = jnp.zeros((nb, nb), jnp.float32)
    for t in range(max_retain):
        sel = kept * (pos == float(t)).astype(jnp.float32)
        idt = jnp.sum(colf * sel, axis=-1, keepdims=True)
        has = jnp.sum(sel, axis=-1, keepdims=True)
        val = jnp.where(has > 0.0, idt, lastid)
        meta = meta + val * (col_ids == t).astype(jnp.float32)
    meta = meta + kk * (col_ids == nb - 1).astype(jnp.float32)
    meta_ref[0] = meta.astype(jnp.int32)


def _attn_body(meta_smem, q_ref, k_ref, v_ref, out_ref, *, nb, block, mr):
    h = pl.program_id(0)
    i = pl.program_id(1)
    base = (h * nb + i) * nb
    cnt = meta_smem[base + nb - 1]
    qb = q_ref[0]

    bids, ss, ms = [], [], []
    for t in range(mr):
        bid = meta_smem[base + t]
        bids.append(bid)
        kt = k_ref[0, pl.ds(bid * block, block), :]
        s = jax.lax.dot_general(qb, kt, (((1,), (1,)), ((), ())),
                                preferred_element_type=jnp.float32)
        ss.append(s)
        ms.append(jnp.max(s, axis=-1, keepdims=True))

    mm = ms
    while len(mm) > 1:
        mm = [jnp.maximum(mm[a], mm[a + 1]) if a + 1 < len(mm) else mm[a]
              for a in range(0, len(mm), 2)]
    M = mm[0]

    l = None
    outs = []
    for t in range(mr):
        gate = jnp.where(t < cnt, 1.0, 0.0)
        e = jnp.exp(ss[t] - M) * gate
        lt = jnp.sum(e, axis=-1, keepdims=True)
        l = lt if l is None else l + lt
        vt = v_ref[0, pl.ds(bids[t] * block, block), :]
        outs.append(jax.lax.dot_general(e.astype(jnp.bfloat16), vt,
                                        (((1,), (0,)), ((), ())),
                                        preferred_element_type=jnp.float32))
    while len(outs) > 1:
        outs = [outs[a] + outs[a + 1] if a + 1 < len(outs) else outs[a]
                for a in range(0, len(outs), 2)]

    out_ref[0] = outs[0] / l


@jax.jit
def kernel(q, k, v):
    B, H, S, d = q.shape
    nb = S // BLOCK
    BH = B * H
    scale = 1.0 / math.sqrt(d)
    min_retain = max(1, int(nb * 0.05))
    max_retain = max(1, int(nb * 0.7))

    qf = q.reshape(BH, S, d)
    kf = k.reshape(BH, S, d)
    vf = v.reshape(BH, S, d)

    meta = pl.pallas_call(
        functools.partial(_mask_body, nb=nb, block=BLOCK, scale=scale,
                          min_retain=min_retain, max_retain=max_retain),
        grid=(BH,),
        in_specs=[
            pl.BlockSpec((1, S, d), lambda h: (h, 0, 0)),
            pl.BlockSpec((1, S, d), lambda h: (h, 0, 0)),
        ],
        out_specs=pl.BlockSpec((1, nb, nb), lambda h: (h, 0, 0)),
        out_shape=jax.ShapeDtypeStruct((BH, nb, nb), jnp.int32),
        compiler_params=pltpu.CompilerParams(
            dimension_semantics=("arbitrary",)),
    )(qf, kf)

    meta_flat = meta.reshape(-1)
    qs = (qf * scale).astype(jnp.bfloat16)
    kb = kf.astype(jnp.bfloat16)
    vb = vf.astype(jnp.bfloat16)

    grid_spec = pltpu.PrefetchScalarGridSpec(
        num_scalar_prefetch=1,
        grid=(BH, nb),
        in_specs=[
            pl.BlockSpec((1, BLOCK, d), lambda h, i, meta: (h, i, 0)),
            pl.BlockSpec((1, S, d), lambda h, i, meta: (h, 0, 0)),
            pl.BlockSpec((1, S, d), lambda h, i, meta: (h, 0, 0)),
        ],
        out_specs=pl.BlockSpec((1, BLOCK, d), lambda h, i, meta: (h, i, 0)),
    )
    out = pl.pallas_call(
        functools.partial(_attn_body, nb=nb, block=BLOCK, mr=max_retain),
        grid_spec=grid_spec,
        out_shape=jax.ShapeDtypeStruct((BH, S, d), jnp.float32),
        compiler_params=pltpu.CompilerParams(
            dimension_semantics=("parallel", "arbitrary")),
    )(meta_flat, qs, kb, vb)

    return out.reshape(B, H, S, d)
